# baseline (speedup 1.0000x reference)
import functools
import jax
import jax.numpy as jnp
from jax import lax
from jax.experimental import pallas as pl
from jax.experimental.pallas import tpu as pltpu
from jax.experimental.pallas import tpu_sc as plsc

N_NODES = 10000
N_EDGES = 320000
N_SUB = 20000
NHID = 128
P_TOT = 1024
B = 32
NPATCH = 32
NTGT = 4
PRW = 16
NHEADS = 4

NC = 2
NS = 16
NW = NC * NS
LANES = 16
HH = 64

NP_SUB = 20224
SB = 1088
SN = 10064
SS = 20224

_mesh = functools.partial(plsc.VectorSubcoreMesh,
                          core_axis_name="c", subcore_axis_name="s")

_SC_PARAMS = pltpu.CompilerParams(use_tc_tiling_on_sc=False,
                                  needs_layout_passes=False)


def _gather_half_body(make_load):
    def body_core(rows_ps, nfull, rem, C, s, idx_ref, tbl_ref, out_ref,
                  idx_v, buf, sem, rembufs):
        base0 = s * rows_ps

        def step(k, _):
            base = base0 + k * C
            pltpu.sync_copy(idx_ref.at[pl.ds(base, C)], idx_v)
            make_load(base, C, tbl_ref, idx_v, buf, sem)
            pltpu.sync_copy(buf, out_ref.at[pl.ds(base, C)])
            return 0

        lax.fori_loop(0, nfull, step, 0)
        if rem:
            idx_r, buf_r = rembufs
            base = base0 + nfull * C
            pltpu.sync_copy(idx_ref.at[pl.ds(base, rem)], idx_r)
            make_load(base, rem, tbl_ref, idx_r, buf_r, sem)
            pltpu.sync_copy(buf_r, out_ref.at[pl.ds(base, rem)])
    return body_core


def _sc_gather_half(tbl_lo, tbl_hi, idx):
    N = idx.shape[0]
    rows_ps = N // NS
    C = 128
    nfull, rem = divmod(rows_ps, C)
    scratch = [pltpu.VMEM((C,), jnp.int32),
               pltpu.VMEM((C, HH), jnp.float32),
               pltpu.SemaphoreType.DMA]
    if rem:
        scratch += [pltpu.VMEM((rem,), jnp.int32),
                    pltpu.VMEM((rem, HH), jnp.float32)]

    def load(base, n, tbl_ref, idx_v, buf, sem):
        pltpu.async_copy(tbl_ref.at[idx_v], buf, sem).wait()

    core_body = _gather_half_body(load)

    def body(tl, th, idx_ref, ol, oh, idx_v, buf, sem, *rembufs):
        c = lax.axis_index("c")
        s = lax.axis_index("s")

        @pl.when(c == 0)
        def _lo():
            core_body(rows_ps, nfull, rem, C, s, idx_ref, tl, ol,
                      idx_v, buf, sem, rembufs)

        @pl.when(c == 1)
        def _hi():
            core_body(rows_ps, nfull, rem, C, s, idx_ref, th, oh,
                      idx_v, buf, sem, rembufs)

    fn = pl.kernel(body,
                   out_type=[jax.ShapeDtypeStruct((N, HH), jnp.float32),
                             jax.ShapeDtypeStruct((N, HH), jnp.float32)],
                   mesh=_mesh(),
                   compiler_params=_SC_PARAMS,
                   scratch_types=scratch)
    return fn(tbl_lo, tbl_hi, idx)


def _sc_mix_scatter(R_lo, R_hi, gidx, h_lo, h_hi, sidx, S):
    N = gidx.shape[0]
    rows_ps = N // NS
    C = 128
    nfull, rem = divmod(rows_ps, C)
    so = S // NS
    scratch = [pltpu.VMEM((C,), jnp.int32),
               pltpu.VMEM((C,), jnp.int32),
               pltpu.VMEM((C, HH), jnp.float32),
               pltpu.VMEM((16, HH), jnp.float32),
               pltpu.VMEM_SHARED((S, HH), jnp.float32),
               pltpu.SemaphoreType.DMA]
    if rem:
        scratch += [pltpu.VMEM((rem,), jnp.int32),
                    pltpu.VMEM((rem,), jnp.int32),
                    pltpu.VMEM((rem, HH), jnp.float32)]

    def body(rl, rh, gidx_ref, hl, hh_, sidx_ref, out_ref,
             gi, siv, buf, zbuf, acc, sem, *rembufs):
        c = lax.axis_index("c")
        s = lax.axis_index("s")
        _zero_zbuf(zbuf, HH)
        _zero_acc(zbuf, acc, s, so)
        plsc.subcore_barrier()

        base0 = s * rows_ps

        def run(r_ref, h_ref):
            def chunk(base, n, giv, svv, bf):
                pltpu.sync_copy(gidx_ref.at[pl.ds(base, n)], giv)
                pltpu.sync_copy(sidx_ref.at[pl.ds(base, n)], svv)
                pltpu.sync_copy(h_ref.at[pl.ds(base, n)], bf)
                pltpu.async_copy(r_ref.at[giv], bf, sem, add=True).wait()
                pltpu.sync_copy(bf, acc.at[svv], add=True)

            def step(k, _):
                chunk(base0 + k * C, C, gi, siv, buf)
                return 0

            lax.fori_loop(0, nfull, step, 0)
            if rem:
                gr, sr, br = rembufs
                chunk(base0 + nfull * C, rem, gr, sr, br)

        @pl.when(c == 0)
        def _lo():
            run(rl, hl)

        @pl.when(c == 1)
        def _hi():
            run(rh, hh_)

        plsc.subcore_barrier()
        pltpu.sync_copy(acc.at[pl.ds(s * so, so)],
                        out_ref.at[pl.ds(s * so, so), pl.ds(c * HH, HH)])

    fn = pl.kernel(body,
                   out_type=jax.ShapeDtypeStruct((S, NHID), jnp.float32),
                   mesh=_mesh(),
                   compiler_params=_SC_PARAMS,
                   scratch_types=scratch)
    return fn(R_lo, R_hi, gidx, h_lo, h_hi, sidx)


def _zero_zbuf(zbuf, H):
    iota = lax.iota(jnp.int32, LANES)

    def zrow(r, _):
        rfull = jnp.full((LANES,), r, jnp.int32)
        for cc in range(H // LANES):
            plsc.store_scatter(zbuf, [rfull, iota + cc * LANES],
                               jnp.zeros((LANES,), jnp.float32))
        return 0

    lax.fori_loop(0, 16, zrow, 0)


def _zero_acc(zbuf, acc, s, so):
    nz, zr = divmod(so, 16)

    def zblk(bk, _):
        pltpu.sync_copy(zbuf, acc.at[pl.ds(s * so + bk * 16, 16)])
        return 0

    lax.fori_loop(0, nz, zblk, 0)
    if zr:
        pltpu.sync_copy(zbuf.at[pl.ds(0, zr)],
                        acc.at[pl.ds(s * so + nz * 16, zr)])


def _sc_scatter_add_split(d_lo, d_hi, idx, S):
    N = idx.shape[0]
    rows_ps = N // NS
    C = 128
    nfull, rem = divmod(rows_ps, C)
    so = S // NS
    scratch = [pltpu.VMEM((C,), jnp.int32),
               pltpu.VMEM((C, HH), jnp.float32),
               pltpu.VMEM((16, HH), jnp.float32),
               pltpu.VMEM_SHARED((S, HH), jnp.float32)]
    if rem:
        scratch += [pltpu.VMEM((rem,), jnp.int32),
                    pltpu.VMEM((rem, HH), jnp.float32)]

    def body(dl, dh, idx_ref, out_ref, idx_v, buf, zbuf, acc, *rembufs):
        c = lax.axis_index("c")
        s = lax.axis_index("s")
        _zero_zbuf(zbuf, HH)
        _zero_acc(zbuf, acc, s, so)
        plsc.subcore_barrier()

        base0 = s * rows_ps

        def run(data_ref):
            def step(k, _):
                base = base0 + k * C
                pltpu.sync_copy(idx_ref.at[pl.ds(base, C)], idx_v)
                pltpu.sync_copy(data_ref.at[pl.ds(base, C)], buf)
                pltpu.sync_copy(buf, acc.at[idx_v], add=True)
                return 0

            lax.fori_loop(0, nfull, step, 0)
            if rem:
                idx_r, buf_r = rembufs
                base = base0 + nfull * C
                pltpu.sync_copy(idx_ref.at[pl.ds(base, rem)], idx_r)
                pltpu.sync_copy(data_ref.at[pl.ds(base, rem)], buf_r)
                pltpu.sync_copy(buf_r, acc.at[idx_r], add=True)

        @pl.when(c == 0)
        def _lo():
            run(dl)

        @pl.when(c == 1)
        def _hi():
            run(dh)

        plsc.subcore_barrier()
        pltpu.sync_copy(acc.at[pl.ds(s * so, so)],
                        out_ref.at[pl.ds(s * so, so), pl.ds(c * HH, HH)])

    fn = pl.kernel(body,
                   out_type=jax.ShapeDtypeStruct((S, NHID), jnp.float32),
                   mesh=_mesh(),
                   compiler_params=_SC_PARAMS,
                   scratch_types=scratch)
    return fn(d_lo, d_hi, idx)


def _sc_gine_msg(e_lo, e_hi, h_lo, h_hi, em, src, dst):
    SCK = 128
    rows_ps = N_EDGES // NS
    nsup, rem = divmod(rows_ps, SCK)
    ntrip = (nsup + 2) // 3
    so = SS // NS
    nslot = 3
    scratch = ([pltpu.VMEM((SCK, HH), jnp.float32)] * nslot
               + [pltpu.VMEM((SCK,), jnp.int32)] * (3 * nslot)
               + [pltpu.VMEM((16, HH), jnp.float32),
                  pltpu.VMEM_SHARED((SS, HH), jnp.float32)]
               + [pltpu.SemaphoreType.DMA] * (3 * nslot))
    if rem:
        scratch += [pltpu.VMEM((rem, HH), jnp.float32),
                    pltpu.VMEM((rem,), jnp.int32),
                    pltpu.VMEM((rem,), jnp.int32),
                    pltpu.VMEM((rem,), jnp.int32)]

    def body(el, eh, hl, hh_, em_ref, src_ref, dst_ref, al, ah, *rest):
        (eb0, eb1, eb2, ei0, ei1, ei2, si0, si1, si2, di0, di1, di2,
         zbuf, acc, lsem0, lsem1, lsem2, esem0, esem1, esem2,
         gsem0, gsem1, gsem2, *rembufs) = rest
        c = lax.axis_index("c")
        s = lax.axis_index("s")
        _zero_zbuf(zbuf, HH)
        _zero_acc(zbuf, acc, s, so)
        plsc.subcore_barrier()

        base0 = s * rows_ps
        ebufs = (eb0, eb1, eb2)
        eidxs = (ei0, ei1, ei2)
        sidxs = (si0, si1, si2)
        didxs = (di0, di1, di2)
        lsems = (lsem0, lsem1, lsem2)
        esems = (esem0, esem1, esem2)
        gsems = (gsem0, gsem1, gsem2)

        def run(e_ref, h_ref):
            def issue(k, slot):
                base = base0 + k * SCK
                pltpu.async_copy(em_ref.at[pl.ds(base, SCK)], eidxs[slot],
                                 lsems[slot])
                pltpu.async_copy(src_ref.at[pl.ds(base, SCK)], sidxs[slot],
                                 lsems[slot])
                pltpu.async_copy(dst_ref.at[pl.ds(base, SCK)], didxs[slot],
                                 lsems[slot])

            def drain_loads(k, slot):
                base = base0 + k * SCK
                pltpu.make_async_copy(em_ref.at[pl.ds(base, SCK)],
                                      eidxs[slot], lsems[slot]).wait()
                pltpu.make_async_copy(src_ref.at[pl.ds(base, SCK)],
                                      sidxs[slot], lsems[slot]).wait()
                pltpu.make_async_copy(dst_ref.at[pl.ds(base, SCK)],
                                      didxs[slot], lsems[slot]).wait()

            def fire_egather(slot):
                pltpu.async_copy(e_ref.at[eidxs[slot]], ebufs[slot],
                                 esems[slot])

            def wait_egather(slot):
                pltpu.make_async_copy(e_ref.at[eidxs[slot]], ebufs[slot],
                                      esems[slot]).wait()

            def fire_hgather(slot):
                pltpu.async_copy(h_ref.at[sidxs[slot]], ebufs[slot],
                                 gsems[slot], add=True)

            def finish(slot):
                pltpu.make_async_copy(h_ref.at[sidxs[slot]], ebufs[slot],
                                      gsems[slot]).wait()
                eb = ebufs[slot]

                def rrow(j, _):
                    for kk in range(HH // LANES):
                        v = eb[j, pl.ds(kk * LANES, LANES)]
                        eb[j, pl.ds(kk * LANES, LANES)] = jnp.maximum(v, 0.0)
                    return 0

                lax.fori_loop(0, SCK, rrow, 0)
                pltpu.sync_copy(eb, acc.at[didxs[slot]], add=True)

            def step(k, slot):
                wait_egather(slot)
                fire_hgather(slot)

                @pl.when(k + 2 < nsup)
                def _pref():
                    issue(k + 2, (slot + 2) % 3)

                @pl.when(k + 1 < nsup)
                def _g():
                    drain_loads(k + 1, (slot + 1) % 3)
                    fire_egather((slot + 1) % 3)

                finish(slot)

            issue(0, 0)
            issue(1, 1)
            drain_loads(0, 0)
            fire_egather(0)

            def trip(g, _):
                k0 = g * 3
                step(k0, 0)

                @pl.when(k0 + 1 < nsup)
                def _s1():
                    step(k0 + 1, 1)

                @pl.when(k0 + 2 < nsup)
                def _s2():
                    step(k0 + 2, 2)

                return 0

            lax.fori_loop(0, ntrip, trip, 0)
            if rem:
                ebr, emr, sir, dir_ = rembufs
                base = base0 + nsup * SCK
                pltpu.sync_copy(em_ref.at[pl.ds(base, rem)], emr)
                pltpu.sync_copy(src_ref.at[pl.ds(base, rem)], sir)
                pltpu.sync_copy(dst_ref.at[pl.ds(base, rem)], dir_)
                pltpu.async_copy(e_ref.at[emr], ebr, esem0).wait()
                pltpu.async_copy(h_ref.at[sir], ebr, gsem0, add=True).wait()

                def rrow2(j, _):
                    for kk in range(HH // LANES):
                        v = ebr[j, pl.ds(kk * LANES, LANES)]
                        ebr[j, pl.ds(kk * LANES, LANES)] = jnp.maximum(v, 0.0)
                    return 0

                lax.fori_loop(0, rem, rrow2, 0)
                pltpu.sync_copy(ebr, acc.at[dir_], add=True)

        @pl.when(c == 0)
        def _lo():
            run(el, hl)

        @pl.when(c == 1)
        def _hi():
            run(eh, hh_)

        plsc.subcore_barrier()

        @pl.when(c == 0)
        def _wlo():
            pltpu.sync_copy(acc.at[pl.ds(s * so, so)],
                            al.at[pl.ds(s * so, so)])

        @pl.when(c == 1)
        def _whi():
            pltpu.sync_copy(acc.at[pl.ds(s * so, so)],
                            ah.at[pl.ds(s * so, so)])

    fn = pl.kernel(body,
                   out_type=[jax.ShapeDtypeStruct((SS, HH), jnp.float32),
                             jax.ShapeDtypeStruct((SS, HH), jnp.float32)],
                   mesh=_mesh(),
                   compiler_params=_SC_PARAMS,
                   scratch_types=scratch)
    return fn(e_lo, e_hi, h_lo, h_hi, em, src, dst)


def _sc_counts(idxb, idxn):
    N = idxb.shape[0]
    H = 16
    rows_pw = N // NW
    C = 128
    nfull, rem = divmod(rows_pw, C)
    sob = SB // NS
    son = SN // NS
    scratch = [pltpu.VMEM((C,), jnp.int32),
               pltpu.VMEM((C, H), jnp.float32),
               pltpu.VMEM((16, H), jnp.float32),
               pltpu.VMEM_SHARED((SB, H), jnp.float32),
               pltpu.VMEM_SHARED((SN, H), jnp.float32)]
    if rem:
        scratch += [pltpu.VMEM((rem,), jnp.int32)]

    def body(idxb_ref, idxn_ref, outb_ref, outn_ref,
             idx_v, ones, zbuf, accb, accn, *rembufs):
        c = lax.axis_index("c")
        s = lax.axis_index("s")
        iota = lax.iota(jnp.int32, LANES)
        one = jnp.ones((LANES,), jnp.float32)

        def orow(r, _):
            plsc.store_scatter(ones, [jnp.full((LANES,), r, jnp.int32), iota],
                               one)
            return 0

        lax.fori_loop(0, C, orow, 0)
        _zero_zbuf(zbuf, H)
        _zero_acc(zbuf, accb, s, sob)
        _zero_acc(zbuf, accn, s, son)
        plsc.subcore_barrier()

        w = s * NC + c
        base0 = w * rows_pw

        def step(k, _):
            base = base0 + k * C
            pltpu.sync_copy(idxb_ref.at[pl.ds(base, C)], idx_v)
            pltpu.sync_copy(ones, accb.at[idx_v], add=True)
            pltpu.sync_copy(idxn_ref.at[pl.ds(base, C)], idx_v)
            pltpu.sync_copy(ones, accn.at[idx_v], add=True)
            return 0

        lax.fori_loop(0, nfull, step, 0)
        if rem:
            (idx_r,) = rembufs
            base = base0 + nfull * C
            pltpu.sync_copy(idxb_ref.at[pl.ds(base, rem)], idx_r)
            pltpu.sync_copy(ones.at[pl.ds(0, rem)], accb.at[idx_r], add=True)
            pltpu.sync_copy(idxn_ref.at[pl.ds(base, rem)], idx_r)
            pltpu.sync_copy(ones.at[pl.ds(0, rem)], accn.at[idx_r], add=True)
        plsc.subcore_barrier()
        pltpu.sync_copy(accb.at[pl.ds(s * sob, sob)],
                        outb_ref.at[c, pl.ds(s * sob, sob)])
        pltpu.sync_copy(accn.at[pl.ds(s * son, son)],
                        outn_ref.at[c, pl.ds(s * son, son)])

    fn = pl.kernel(body,
                   out_type=[jax.ShapeDtypeStruct((NC, SB, H), jnp.float32),
                             jax.ShapeDtypeStruct((NC, SN, H), jnp.float32)],
                   mesh=_mesh(),
                   compiler_params=_SC_PARAMS,
                   scratch_types=scratch)
    return fn(idxb, idxn)


def _sc_seg_max(table, gidx, ids, S):
    N = gidx.shape[0]
    H = 16
    rows_pw = N // NW
    C = 128
    nfull, rem = divmod(rows_pw, C)
    scratch = [pltpu.VMEM((C,), jnp.int32),
               pltpu.VMEM((C,), jnp.int32),
               pltpu.VMEM((C, H), jnp.float32),
               pltpu.VMEM((S, H), jnp.float32),
               pltpu.SemaphoreType.DMA]
    if rem:
        scratch += [pltpu.VMEM((rem,), jnp.int32),
                    pltpu.VMEM((rem,), jnp.int32),
                    pltpu.VMEM((rem, H), jnp.float32)]

    def body(table_ref, gidx_ref, ids_ref, out_ref,
             gidx_v, ids_v, rows_v, tbl, sem, *rembufs):
        w = lax.axis_index("s") * NC + lax.axis_index("c")
        iota = lax.iota(jnp.int32, LANES)
        ninf = jnp.full((LANES,), -jnp.inf, jnp.float32)

        def init_row(r, _):
            plsc.store_scatter(tbl, [jnp.full((LANES,), r, jnp.int32), iota],
                               ninf)
            return 0

        lax.fori_loop(0, S, init_row, 0)

        def do_chunk(base, n, gv, iv, rv):
            pltpu.sync_copy(gidx_ref.at[pl.ds(base, n)], gv)
            pltpu.sync_copy(ids_ref.at[pl.ds(base, n)], iv)
            pltpu.async_copy(table_ref.at[gv], rv, sem).wait()

            def row(j, _):
                jfull = jnp.full((LANES,), j, jnp.int32)
                seg = plsc.load_gather(iv, [jfull])
                val = plsc.load_gather(rv, [jfull, iota])
                cur = plsc.load_gather(tbl, [seg, iota])
                plsc.store_scatter(tbl, [seg, iota], jnp.maximum(cur, val))
                return 0

            lax.fori_loop(0, n, row, 0)

        base0 = w * rows_pw

        def step(k, _):
            do_chunk(base0 + k * C, C, gidx_v, ids_v, rows_v)
            return 0

        lax.fori_loop(0, nfull, step, 0)
        if rem:
            gr, ir, rr = rembufs
            do_chunk(base0 + nfull * C, rem, gr, ir, rr)
        pltpu.sync_copy(tbl, out_ref.at[w])

    fn = pl.kernel(body,
                   out_type=jax.ShapeDtypeStruct((NW, S, H), jnp.float32),
                   mesh=_mesh(),
                   compiler_params=_SC_PARAMS,
                   scratch_types=scratch)
    return fn(table, gidx, ids)


def _full_spec(shape):
    nd = len(shape)
    return pl.BlockSpec(shape, lambda *_: (0,) * nd)


def _tc_linear_split(x, W, b, T):
    N, K = x.shape
    grid = (N // T,)

    def body(x_ref, w_ref, b_ref, ol, oh):
        r = jnp.dot(x_ref[...], w_ref[...],
                    preferred_element_type=jnp.float32) + b_ref[...]
        ol[...] = r[:, :HH]
        oh[...] = r[:, HH:]

    return pl.pallas_call(
        body, grid=grid,
        in_specs=[pl.BlockSpec((T, K), lambda i: (i, 0)),
                  _full_spec((K, NHID)), _full_spec((1, NHID))],
        out_specs=[pl.BlockSpec((T, HH), lambda i: (i, 0)),
                   pl.BlockSpec((T, HH), lambda i: (i, 0))],
        out_shape=[jax.ShapeDtypeStruct((N, HH), jnp.float32),
                   jax.ShapeDtypeStruct((N, HH), jnp.float32)])(x, W, b)


def _tc_gine_split(h_lo, h_hi, a_lo, a_hi, W1, b1, W2, b2):
    N = h_lo.shape[0]
    T = 632
    grid = (N // T,)

    def body(hl, hh_, al, ah, w1, b1_, w2, b2_, ol, oh):
        hv = jnp.concatenate([hl[...], hh_[...]], axis=1)
        u = hv + jnp.concatenate([al[...], ah[...]], axis=1)
        t = jnp.maximum(
            jnp.dot(u, w1[...], preferred_element_type=jnp.float32)
            + b1_[...], 0.0)
        v = jnp.dot(t, w2[...], preferred_element_type=jnp.float32) + b2_[...]
        r = jnp.maximum(v, 0.0) + hv
        ol[...] = r[:, :HH]
        oh[...] = r[:, HH:]

    rs = pl.BlockSpec((T, HH), lambda i: (i, 0))
    return pl.pallas_call(
        body, grid=grid,
        in_specs=[rs, rs, rs, rs,
                  _full_spec((NHID, NHID)), _full_spec((1, NHID)),
                  _full_spec((NHID, NHID)), _full_spec((1, NHID))],
        out_specs=[rs, rs],
        out_shape=[jax.ShapeDtypeStruct((N, HH), jnp.float32),
                   jax.ShapeDtypeStruct((N, HH), jnp.float32)])(
            h_lo, h_hi, a_lo, a_hi, W1, b1, W2, b2)


def _tc_mean_lin_relu_split(Ssum, cparts, W, b):
    S = Ssum.shape[0]

    def body(s_ref, c_ref, w_ref, b_ref, ol, oh):
        cp = c_ref[...]
        cnt = cp[0, :, :1] + cp[1, :, :1]
        m = s_ref[...] / jnp.maximum(cnt, 1.0)
        r = jnp.maximum(
            jnp.dot(m, w_ref[...], preferred_element_type=jnp.float32)
            + b_ref[...], 0.0)
        ol[...] = r[:, :HH]
        oh[...] = r[:, HH:]

    return pl.pallas_call(
        body,
        in_specs=[_full_spec((S, NHID)), _full_spec((NC, S, PRW)),
                  _full_spec((NHID, NHID)), _full_spec((1, NHID))],
        out_specs=[_full_spec((S, HH)), _full_spec((S, HH))],
        out_shape=[jax.ShapeDtypeStruct((S, HH), jnp.float32),
                   jax.ShapeDtypeStruct((S, HH), jnp.float32)])(
            Ssum, cparts, W, b)


def _tc_mean_split(Ssum, cparts):
    S = Ssum.shape[0]

    def body(s_ref, c_ref, ol, oh):
        cp = c_ref[...]
        cnt = cp[0, :, :1] + cp[1, :, :1]
        r = s_ref[...] / jnp.maximum(cnt, 1.0)
        ol[...] = r[:, :HH]
        oh[...] = r[:, HH:]

    return pl.pallas_call(
        body,
        in_specs=[_full_spec((S, NHID)), _full_spec((NC, S, PRW))],
        out_specs=[_full_spec((S, HH)), _full_spec((S, HH))],
        out_shape=[jax.ShapeDtypeStruct((S, HH), jnp.float32),
                   jax.ShapeDtypeStruct((S, HH), jnp.float32)])(Ssum, cparts)


def _ln(h, s, bb):
    m = h.mean(-1, keepdims=True)
    v = ((h - m) ** 2).mean(-1, keepdims=True)
    return (h - m) / jnp.sqrt(v + 1e-5) * s + bb


def _tc_tail(S3, cb2, pesp, ctx_ids, tgt_ids, flat):
    BT = B * NTGT

    def body(*refs):
        (s3, c2, pp, cid, tid,
         peW, peb,
         cWv, cbv, cWo, cbo, cf1, cb1, cf2, cb2_, cl1s, cl1b, cl2s, cl2b,
         tWq, tbq, tWk, tbk, tWv, tbv, tWo, tbo, tf1, tb1, tf2, tb2_,
         tl1s, tl1b, tl2s, tl2b,
         p1W, p1b, p2W, p2b, p3W, p3b,
         o1, o2) = refs

        cp = c2[...]
        cnt = cp[0, :, :1] + cp[1, :, :1]
        M3 = (s3[...] / jnp.maximum(cnt, 1.0))[:P_TOT]
        pmax = jnp.max(pp[...], axis=0)
        P = jnp.where(jnp.isfinite(pmax), pmax, 0.0)[:P_TOT]

        cidv = cid[...].reshape(B, 1)
        tidv = tid[...].reshape(BT, 1)
        oh_c = (lax.broadcasted_iota(jnp.int32, (B, P_TOT), 1)
                == cidv).astype(jnp.float32)
        oh_t = (lax.broadcasted_iota(jnp.int32, (BT, P_TOT), 1)
                == tidv).astype(jnp.float32)

        dotf = functools.partial(jnp.dot, preferred_element_type=jnp.float32)
        dotx = functools.partial(jnp.dot, preferred_element_type=jnp.float32,
                                 precision=lax.Precision.HIGHEST)
        cx = dotx(oh_c, M3)
        tx = dotx(oh_t, M3)
        cpe = dotx(oh_c, P)
        tpes = dotx(oh_t, P)

        cx = cx + dotf(cpe, peW[...]) + peb[...]
        enc_t = dotf(tpes, peW[...]) + peb[...]

        v = dotf(cx, cWv[...]) + cbv[...]
        cx = _ln(cx + dotf(v, cWo[...]) + cbo[...], cl1s[...], cl1b[...])
        f = dotf(jnp.maximum(dotf(cx, cf1[...]) + cb1[...], 0.0),
                 cf2[...]) + cb2_[...]
        cx = _ln(cx + f, cl2s[...], cl2b[...])

        q = dotf(tx, tWq[...]) + tbq[...]
        k = dotf(tx, tWk[...]) + tbk[...]
        vv = dotf(tx, tWv[...]) + tbv[...]
        ri = lax.broadcasted_iota(jnp.int32, (BT, BT), 0) // NTGT
        ci = lax.broadcasted_iota(jnp.int32, (BT, BT), 1) // NTGT
        blk = ri == ci
        dh = NHID // NHEADS
        outs = []
        for hd in range(NHEADS):
            sl = slice(hd * dh, (hd + 1) * dh)
            A = lax.dot_general(q[:, sl], k[:, sl],
                                (((1,), (1,)), ((), ())),
                                preferred_element_type=jnp.float32)
            A = A / jnp.sqrt(jnp.float32(dh))
            A = jnp.where(blk, A, -1e9)
            A = A - jnp.max(A, axis=-1, keepdims=True)
            E = jnp.exp(A)
            Pm = E / jnp.sum(E, axis=-1, keepdims=True)
            outs.append(dotf(Pm, vv[:, sl]))
        o = jnp.concatenate(outs, axis=1)
        tx = _ln(tx + dotf(o, tWo[...]) + tbo[...], tl1s[...], tl1b[...])
        f = dotf(jnp.maximum(dotf(tx, tf1[...]) + tb1[...], 0.0),
                 tf2[...]) + tb2_[...]
        tx = _ln(tx + f, tl2s[...], tl2b[...])

        m = jnp.mean(tx, axis=-1, keepdims=True)
        em, emi = jnp.exp(m), jnp.exp(-m)
        xo = jnp.concatenate([(em + emi) * 0.5, (em - emi) * 0.5], axis=1)
        o1[...] = jnp.concatenate(
            [xo, jnp.zeros((BT, NHID - 2), jnp.float32)], axis=1)

        rep = (lax.broadcasted_iota(jnp.int32, (BT, B), 0) // NTGT
               == lax.broadcasted_iota(jnp.int32, (BT, B), 1)
               ).astype(jnp.float32)
        tpe = dotx(rep, cx) + enc_t
        t1 = jnp.maximum(dotf(tpe, p1W[...]) + p1b[...], 0.0)
        t2 = jnp.maximum(dotf(t1, p2W[...]) + p2b[...], 0.0)
        o2[...] = dotf(t2, p3W[...]) + p3b[...]

    specs = [_full_spec(a.shape) for a in flat]
    return pl.pallas_call(
        body,
        in_specs=[_full_spec(S3.shape), _full_spec(cb2.shape),
                  _full_spec(pesp.shape), _full_spec(ctx_ids.shape),
                  _full_spec(tgt_ids.shape)] + specs,
        out_specs=[_full_spec((BT, NHID)), _full_spec((BT, NHID))],
        out_shape=[jax.ShapeDtypeStruct((BT, NHID), jnp.float32),
                   jax.ShapeDtypeStruct((BT, NHID), jnp.float32)])(
            S3, cb2, pesp, ctx_ids, tgt_ids, *flat)


def _rb(b):
    return b.reshape(1, -1)


def kernel(x, edge_attr, rw_pos_enc, combined_subgraphs,
           subgraphs_nodes_mapper, subgraphs_edges_mapper, subgraphs_batch,
           context_subgraph_idx, target_subgraph_idxs, mask, params):
    p = params
    i32 = jnp.int32
    nm = subgraphs_nodes_mapper.astype(i32)
    em = subgraphs_edges_mapper.astype(i32)
    bx = subgraphs_batch.astype(i32)
    src = combined_subgraphs[0].astype(i32)
    dst = combined_subgraphs[1].astype(i32)

    padn = NP_SUB - N_SUB
    nm_g = jnp.concatenate([nm, jnp.zeros((padn,), i32)])
    nm_s = jnp.concatenate([nm, jnp.full((padn,), N_NODES, i32)])
    bx_s = jnp.concatenate([bx, jnp.full((padn,), P_TOT, i32)])
    h0_lo, h0_hi = _tc_linear_split(x, p["inp"]["W"], _rb(p["inp"]["b"]),
                                    2000)
    e_lo, e_hi = _tc_linear_split(edge_attr, p["edge"]["W"],
                                  _rb(p["edge"]["b"]), 512)
    h_lo, h_hi = _sc_gather_half(h0_lo, h0_hi, nm_g)
    pesp = _sc_seg_max(rw_pos_enc, nm_g, bx_s, SB)
    cb2, cn2 = _sc_counts(bx_s, nm_s)

    a_lo, a_hi = _sc_gine_msg(e_lo, e_hi, h_lo, h_hi, em, src, dst)
    g = p["gnn"][0]
    h_lo, h_hi = _tc_gine_split(h_lo, h_hi, a_lo, a_hi,
                                g["l1"]["W"], _rb(g["l1"]["b"]),
                                g["l2"]["W"], _rb(g["l2"]["b"]))

    S1 = _sc_scatter_add_split(h_lo, h_hi, bx_s, SB)
    R_lo, R_hi = _tc_mean_lin_relu_split(S1, cb2, p["U"]["W"],
                                         _rb(p["U"]["b"]))
    S2 = _sc_mix_scatter(R_lo, R_hi, bx_s, h_lo, h_hi, nm_s, SN)
    M2_lo, M2_hi = _tc_mean_split(S2, cn2)
    h_lo, h_hi = _sc_gather_half(M2_lo, M2_hi, nm_g)

    a_lo, a_hi = _sc_gine_msg(e_lo, e_hi, h_lo, h_hi, em, src, dst)
    g = p["gnn"][1]
    h_lo, h_hi = _tc_gine_split(h_lo, h_hi, a_lo, a_hi,
                                g["l1"]["W"], _rb(g["l1"]["b"]),
                                g["l2"]["W"], _rb(g["l2"]["b"]))

    S3 = _sc_scatter_add_split(h_lo, h_hi, bx_s, SB)

    batch_indexer = jnp.arange(B, dtype=i32) * NPATCH
    ctx_ids = (context_subgraph_idx.astype(i32) + batch_indexer).reshape(1, B)
    tgt_ids = (target_subgraph_idxs.astype(i32)
               + batch_indexer[:, None]).reshape(1, B * NTGT)

    c = p["ctx"][0]
    t = p["tgt"][0]
    p1, p2, p3 = p["pred"]
    p3W = jnp.zeros((NHID, NHID), jnp.float32).at[:, :2].set(p3["W"])
    p3b = jnp.zeros((1, NHID), jnp.float32).at[:, :2].set(_rb(p3["b"]))
    flat = [
        p["pe"]["W"], _rb(p["pe"]["b"]),
        c["Wv"]["W"], _rb(c["Wv"]["b"]), c["Wo"]["W"], _rb(c["Wo"]["b"]),
        c["ff1"]["W"], _rb(c["ff1"]["b"]), c["ff2"]["W"], _rb(c["ff2"]["b"]),
        _rb(c["ln1"]["s"]), _rb(c["ln1"]["b"]),
        _rb(c["ln2"]["s"]), _rb(c["ln2"]["b"]),
        t["Wq"]["W"], _rb(t["Wq"]["b"]), t["Wk"]["W"], _rb(t["Wk"]["b"]),
        t["Wv"]["W"], _rb(t["Wv"]["b"]), t["Wo"]["W"], _rb(t["Wo"]["b"]),
        t["ff1"]["W"], _rb(t["ff1"]["b"]), t["ff2"]["W"], _rb(t["ff2"]["b"]),
        _rb(t["ln1"]["s"]), _rb(t["ln1"]["b"]),
        _rb(t["ln2"]["s"]), _rb(t["ln2"]["b"]),
        p1["W"], _rb(p1["b"]), p2["W"], _rb(p2["b"]), p3W, p3b,
    ]
    o1, o2 = _tc_tail(S3, cb2, pesp, ctx_ids, tgt_ids, flat)

    target_x_out = o1[:, :2].reshape(B, NTGT, 2)
    target_y = o2[:, :2].reshape(B, NTGT, 2)
    return (target_x_out, target_y)


# speedup vs baseline: 1.0365x; 1.0365x over previous
import functools
import jax
import jax.numpy as jnp
from jax import lax
from jax.experimental import pallas as pl
from jax.experimental.pallas import tpu as pltpu
from jax.experimental.pallas import tpu_sc as plsc

N_NODES = 10000
N_EDGES = 320000
N_SUB = 20000
NHID = 128
P_TOT = 1024
B = 32
NPATCH = 32
NTGT = 4
PRW = 16
NHEADS = 4

NC = 2
NS = 16
NW = NC * NS
LANES = 16
HH = 64

NP_SUB = 20224
SB = 1088
SN = 10064
SS = 20224

_mesh = functools.partial(plsc.VectorSubcoreMesh,
                          core_axis_name="c", subcore_axis_name="s")

_SC_PARAMS = pltpu.CompilerParams(use_tc_tiling_on_sc=False,
                                  needs_layout_passes=False)


def _sc_gather(table, idx, D):
    N = idx.shape[0]
    rows_pw = N // NW
    C = 128
    nfull, rem = divmod(rows_pw, C)
    scratch = [pltpu.VMEM((C,), jnp.int32),
               pltpu.VMEM((C, D), jnp.float32),
               pltpu.SemaphoreType.DMA]
    if rem:
        scratch += [pltpu.VMEM((rem,), jnp.int32),
                    pltpu.VMEM((rem, D), jnp.float32)]

    def body(table_ref, idx_ref, out_ref, idx_v, rows_v, sem, *rembufs):
        w = lax.axis_index("s") * NC + lax.axis_index("c")
        base0 = w * rows_pw

        def step(k, _):
            base = base0 + k * C
            pltpu.sync_copy(idx_ref.at[pl.ds(base, C)], idx_v)
            pltpu.async_copy(table_ref.at[idx_v], rows_v, sem).wait()
            pltpu.sync_copy(rows_v, out_ref.at[pl.ds(base, C)])
            return 0

        lax.fori_loop(0, nfull, step, 0)
        if rem:
            idx_r, rows_r = rembufs
            base = base0 + nfull * C
            pltpu.sync_copy(idx_ref.at[pl.ds(base, rem)], idx_r)
            pltpu.async_copy(table_ref.at[idx_r], rows_r, sem).wait()
            pltpu.sync_copy(rows_r, out_ref.at[pl.ds(base, rem)])

    fn = pl.kernel(body,
                   out_type=jax.ShapeDtypeStruct((N, D), jnp.float32),
                   mesh=_mesh(),
                   compiler_params=_SC_PARAMS,
                   scratch_types=scratch)
    return fn(table, idx)


def _gather_half_body(make_load):
    def body_core(rows_ps, nfull, rem, C, s, idx_ref, tbl_ref, out_ref,
                  idx_v, buf, sem, rembufs):
        base0 = s * rows_ps

        def step(k, _):
            base = base0 + k * C
            pltpu.sync_copy(idx_ref.at[pl.ds(base, C)], idx_v)
            make_load(base, C, tbl_ref, idx_v, buf, sem)
            pltpu.sync_copy(buf, out_ref.at[pl.ds(base, C)])
            return 0

        lax.fori_loop(0, nfull, step, 0)
        if rem:
            idx_r, buf_r = rembufs
            base = base0 + nfull * C
            pltpu.sync_copy(idx_ref.at[pl.ds(base, rem)], idx_r)
            make_load(base, rem, tbl_ref, idx_r, buf_r, sem)
            pltpu.sync_copy(buf_r, out_ref.at[pl.ds(base, rem)])
    return body_core


def _sc_gather_half(tbl_lo, tbl_hi, idx):
    N = idx.shape[0]
    rows_ps = N // NS
    C = 128
    nfull, rem = divmod(rows_ps, C)
    scratch = [pltpu.VMEM((C,), jnp.int32),
               pltpu.VMEM((C, HH), jnp.float32),
               pltpu.SemaphoreType.DMA]
    if rem:
        scratch += [pltpu.VMEM((rem,), jnp.int32),
                    pltpu.VMEM((rem, HH), jnp.float32)]

    def load(base, n, tbl_ref, idx_v, buf, sem):
        pltpu.async_copy(tbl_ref.at[idx_v], buf, sem).wait()

    core_body = _gather_half_body(load)

    def body(tl, th, idx_ref, ol, oh, idx_v, buf, sem, *rembufs):
        c = lax.axis_index("c")
        s = lax.axis_index("s")

        @pl.when(c == 0)
        def _lo():
            core_body(rows_ps, nfull, rem, C, s, idx_ref, tl, ol,
                      idx_v, buf, sem, rembufs)

        @pl.when(c == 1)
        def _hi():
            core_body(rows_ps, nfull, rem, C, s, idx_ref, th, oh,
                      idx_v, buf, sem, rembufs)

    fn = pl.kernel(body,
                   out_type=[jax.ShapeDtypeStruct((N, HH), jnp.float32),
                             jax.ShapeDtypeStruct((N, HH), jnp.float32)],
                   mesh=_mesh(),
                   compiler_params=_SC_PARAMS,
                   scratch_types=scratch)
    return fn(tbl_lo, tbl_hi, idx)


def _sc_mix_scatter(R_lo, R_hi, gidx, h_lo, h_hi, sidx, S):
    N = gidx.shape[0]
    rows_ps = N // NS
    C = 128
    nfull, rem = divmod(rows_ps, C)
    so = S // NS
    scratch = [pltpu.VMEM((C,), jnp.int32),
               pltpu.VMEM((C,), jnp.int32),
               pltpu.VMEM((C, HH), jnp.float32),
               pltpu.VMEM((16, HH), jnp.float32),
               pltpu.VMEM_SHARED((S, HH), jnp.float32),
               pltpu.SemaphoreType.DMA]
    if rem:
        scratch += [pltpu.VMEM((rem,), jnp.int32),
                    pltpu.VMEM((rem,), jnp.int32),
                    pltpu.VMEM((rem, HH), jnp.float32)]

    def body(rl, rh, gidx_ref, hl, hh_, sidx_ref, out_ref,
             gi, siv, buf, zbuf, acc, sem, *rembufs):
        c = lax.axis_index("c")
        s = lax.axis_index("s")
        _zero_zbuf(zbuf, HH)
        _zero_acc(zbuf, acc, s, so)
        plsc.subcore_barrier()

        base0 = s * rows_ps

        def run(r_ref, h_ref):
            def chunk(base, n, giv, svv, bf):
                pltpu.sync_copy(gidx_ref.at[pl.ds(base, n)], giv)
                pltpu.sync_copy(sidx_ref.at[pl.ds(base, n)], svv)
                pltpu.sync_copy(h_ref.at[pl.ds(base, n)], bf)
                pltpu.async_copy(r_ref.at[giv], bf, sem, add=True).wait()
                pltpu.sync_copy(bf, acc.at[svv], add=True)

            def step(k, _):
                chunk(base0 + k * C, C, gi, siv, buf)
                return 0

            lax.fori_loop(0, nfull, step, 0)
            if rem:
                gr, sr, br = rembufs
                chunk(base0 + nfull * C, rem, gr, sr, br)

        @pl.when(c == 0)
        def _lo():
            run(rl, hl)

        @pl.when(c == 1)
        def _hi():
            run(rh, hh_)

        plsc.subcore_barrier()
        pltpu.sync_copy(acc.at[pl.ds(s * so, so)],
                        out_ref.at[pl.ds(s * so, so), pl.ds(c * HH, HH)])

    fn = pl.kernel(body,
                   out_type=jax.ShapeDtypeStruct((S, NHID), jnp.float32),
                   mesh=_mesh(),
                   compiler_params=_SC_PARAMS,
                   scratch_types=scratch)
    return fn(R_lo, R_hi, gidx, h_lo, h_hi, sidx)


def _zero_zbuf(zbuf, H):
    iota = lax.iota(jnp.int32, LANES)

    def zrow(r, _):
        rfull = jnp.full((LANES,), r, jnp.int32)
        for cc in range(H // LANES):
            plsc.store_scatter(zbuf, [rfull, iota + cc * LANES],
                               jnp.zeros((LANES,), jnp.float32))
        return 0

    lax.fori_loop(0, 16, zrow, 0)


def _zero_acc(zbuf, acc, s, so):
    nz, zr = divmod(so, 16)

    def zblk(bk, _):
        pltpu.sync_copy(zbuf, acc.at[pl.ds(s * so + bk * 16, 16)])
        return 0

    lax.fori_loop(0, nz, zblk, 0)
    if zr:
        pltpu.sync_copy(zbuf.at[pl.ds(0, zr)],
                        acc.at[pl.ds(s * so + nz * 16, zr)])


def _sc_scatter_add_split(d_lo, d_hi, idx, S):
    N = idx.shape[0]
    rows_ps = N // NS
    C = 128
    nfull, rem = divmod(rows_ps, C)
    so = S // NS
    scratch = [pltpu.VMEM((C,), jnp.int32),
               pltpu.VMEM((C, HH), jnp.float32),
               pltpu.VMEM((16, HH), jnp.float32),
               pltpu.VMEM_SHARED((S, HH), jnp.float32)]
    if rem:
        scratch += [pltpu.VMEM((rem,), jnp.int32),
                    pltpu.VMEM((rem, HH), jnp.float32)]

    def body(dl, dh, idx_ref, out_ref, idx_v, buf, zbuf, acc, *rembufs):
        c = lax.axis_index("c")
        s = lax.axis_index("s")
        _zero_zbuf(zbuf, HH)
        _zero_acc(zbuf, acc, s, so)
        plsc.subcore_barrier()

        base0 = s * rows_ps

        def run(data_ref):
            def step(k, _):
                base = base0 + k * C
                pltpu.sync_copy(idx_ref.at[pl.ds(base, C)], idx_v)
                pltpu.sync_copy(data_ref.at[pl.ds(base, C)], buf)
                pltpu.sync_copy(buf, acc.at[idx_v], add=True)
                return 0

            lax.fori_loop(0, nfull, step, 0)
            if rem:
                idx_r, buf_r = rembufs
                base = base0 + nfull * C
                pltpu.sync_copy(idx_ref.at[pl.ds(base, rem)], idx_r)
                pltpu.sync_copy(data_ref.at[pl.ds(base, rem)], buf_r)
                pltpu.sync_copy(buf_r, acc.at[idx_r], add=True)

        @pl.when(c == 0)
        def _lo():
            run(dl)

        @pl.when(c == 1)
        def _hi():
            run(dh)

        plsc.subcore_barrier()
        pltpu.sync_copy(acc.at[pl.ds(s * so, so)],
                        out_ref.at[pl.ds(s * so, so), pl.ds(c * HH, HH)])

    fn = pl.kernel(body,
                   out_type=jax.ShapeDtypeStruct((S, NHID), jnp.float32),
                   mesh=_mesh(),
                   compiler_params=_SC_PARAMS,
                   scratch_types=scratch)
    return fn(d_lo, d_hi, idx)


def _sc_gine_msg(e_lo, e_hi, h_lo, h_hi, src, dst):
    SCK = 128
    rows_ps = N_EDGES // NS
    nsup, rem = divmod(rows_ps, SCK)
    ntrip = (nsup + 2) // 3
    so = SS // NS
    nslot = 3
    scratch = ([pltpu.VMEM((SCK, HH), jnp.float32)] * nslot
               + [pltpu.VMEM((SCK,), jnp.int32)] * nslot
               + [pltpu.VMEM((SCK,), jnp.int32)] * nslot
               + [pltpu.VMEM((16, HH), jnp.float32),
                  pltpu.VMEM_SHARED((SS, HH), jnp.float32)]
               + [pltpu.SemaphoreType.DMA] * nslot
               + [pltpu.SemaphoreType.DMA] * nslot)
    if rem:
        scratch += [pltpu.VMEM((rem, HH), jnp.float32),
                    pltpu.VMEM((rem,), jnp.int32),
                    pltpu.VMEM((rem,), jnp.int32)]

    def body(el, eh, hl, hh_, src_ref, dst_ref, al, ah, *rest):
        (eb0, eb1, eb2, si0, si1, si2, di0, di1, di2, zbuf, acc,
         lsem0, lsem1, lsem2, gsem0, gsem1, gsem2, *rembufs) = rest
        c = lax.axis_index("c")
        s = lax.axis_index("s")
        _zero_zbuf(zbuf, HH)
        _zero_acc(zbuf, acc, s, so)
        plsc.subcore_barrier()

        base0 = s * rows_ps
        ebufs = (eb0, eb1, eb2)
        sidxs = (si0, si1, si2)
        didxs = (di0, di1, di2)
        lsems = (lsem0, lsem1, lsem2)
        gsems = (gsem0, gsem1, gsem2)

        def run(e_ref, h_ref):
            def issue(k, slot):
                base = base0 + k * SCK
                pltpu.async_copy(src_ref.at[pl.ds(base, SCK)], sidxs[slot],
                                 lsems[slot])
                pltpu.async_copy(dst_ref.at[pl.ds(base, SCK)], didxs[slot],
                                 lsems[slot])
                pltpu.async_copy(e_ref.at[pl.ds(base, SCK)], ebufs[slot],
                                 lsems[slot])

            def drain_loads(k, slot):
                base = base0 + k * SCK
                pltpu.make_async_copy(src_ref.at[pl.ds(base, SCK)],
                                      sidxs[slot], lsems[slot]).wait()
                pltpu.make_async_copy(dst_ref.at[pl.ds(base, SCK)],
                                      didxs[slot], lsems[slot]).wait()
                pltpu.make_async_copy(e_ref.at[pl.ds(base, SCK)],
                                      ebufs[slot], lsems[slot]).wait()

            def fire_gather(slot):
                pltpu.async_copy(h_ref.at[sidxs[slot]], ebufs[slot],
                                 gsems[slot], add=True)

            def finish(slot):
                pltpu.make_async_copy(h_ref.at[sidxs[slot]], ebufs[slot],
                                      gsems[slot]).wait()
                eb = ebufs[slot]

                def rrow(j, _):
                    for kk in range(HH // LANES):
                        v = eb[j, pl.ds(kk * LANES, LANES)]
                        eb[j, pl.ds(kk * LANES, LANES)] = jnp.maximum(v, 0.0)
                    return 0

                lax.fori_loop(0, SCK, rrow, 0)
                pltpu.sync_copy(eb, acc.at[didxs[slot]], add=True)

            def step(k, slot):
                @pl.when(k + 2 < nsup)
                def _pref():
                    issue(k + 2, (slot + 2) % 3)

                @pl.when(k + 1 < nsup)
                def _g():
                    drain_loads(k + 1, (slot + 1) % 3)
                    fire_gather((slot + 1) % 3)

                finish(slot)

            issue(0, 0)
            issue(1, 1)
            drain_loads(0, 0)
            fire_gather(0)

            def trip(g, _):
                k0 = g * 3
                step(k0, 0)

                @pl.when(k0 + 1 < nsup)
                def _s1():
                    step(k0 + 1, 1)

                @pl.when(k0 + 2 < nsup)
                def _s2():
                    step(k0 + 2, 2)

                return 0

            lax.fori_loop(0, ntrip, trip, 0)
            if rem:
                ebr, sir, dir_ = rembufs
                base = base0 + nsup * SCK
                pltpu.sync_copy(src_ref.at[pl.ds(base, rem)], sir)
                pltpu.sync_copy(dst_ref.at[pl.ds(base, rem)], dir_)
                pltpu.sync_copy(e_ref.at[pl.ds(base, rem)], ebr)
                pltpu.async_copy(h_ref.at[sir], ebr, gsem0, add=True).wait()

                def rrow2(j, _):
                    for kk in range(HH // LANES):
                        v = ebr[j, pl.ds(kk * LANES, LANES)]
                        ebr[j, pl.ds(kk * LANES, LANES)] = jnp.maximum(v, 0.0)
                    return 0

                lax.fori_loop(0, rem, rrow2, 0)
                pltpu.sync_copy(ebr, acc.at[dir_], add=True)

        @pl.when(c == 0)
        def _lo():
            run(el, hl)

        @pl.when(c == 1)
        def _hi():
            run(eh, hh_)

        plsc.subcore_barrier()

        @pl.when(c == 0)
        def _wlo():
            pltpu.sync_copy(acc.at[pl.ds(s * so, so)],
                            al.at[pl.ds(s * so, so)])

        @pl.when(c == 1)
        def _whi():
            pltpu.sync_copy(acc.at[pl.ds(s * so, so)],
                            ah.at[pl.ds(s * so, so)])

    fn = pl.kernel(body,
                   out_type=[jax.ShapeDtypeStruct((SS, HH), jnp.float32),
                             jax.ShapeDtypeStruct((SS, HH), jnp.float32)],
                   mesh=_mesh(),
                   compiler_params=_SC_PARAMS,
                   scratch_types=scratch)
    return fn(e_lo, e_hi, h_lo, h_hi, src, dst)


def _sc_counts(idxb, idxn):
    N = idxb.shape[0]
    H = 16
    rows_pw = N // NW
    C = 128
    nfull, rem = divmod(rows_pw, C)
    sob = SB // NS
    son = SN // NS
    scratch = [pltpu.VMEM((C,), jnp.int32),
               pltpu.VMEM((C, H), jnp.float32),
               pltpu.VMEM((16, H), jnp.float32),
               pltpu.VMEM_SHARED((SB, H), jnp.float32),
               pltpu.VMEM_SHARED((SN, H), jnp.float32)]
    if rem:
        scratch += [pltpu.VMEM((rem,), jnp.int32)]

    def body(idxb_ref, idxn_ref, outb_ref, outn_ref,
             idx_v, ones, zbuf, accb, accn, *rembufs):
        c = lax.axis_index("c")
        s = lax.axis_index("s")
        iota = lax.iota(jnp.int32, LANES)
        one = jnp.ones((LANES,), jnp.float32)

        def orow(r, _):
            plsc.store_scatter(ones, [jnp.full((LANES,), r, jnp.int32), iota],
                               one)
            return 0

        lax.fori_loop(0, C, orow, 0)
        _zero_zbuf(zbuf, H)
        _zero_acc(zbuf, accb, s, sob)
        _zero_acc(zbuf, accn, s, son)
        plsc.subcore_barrier()

        w = s * NC + c
        base0 = w * rows_pw

        def step(k, _):
            base = base0 + k * C
            pltpu.sync_copy(idxb_ref.at[pl.ds(base, C)], idx_v)
            pltpu.sync_copy(ones, accb.at[idx_v], add=True)
            pltpu.sync_copy(idxn_ref.at[pl.ds(base, C)], idx_v)
            pltpu.sync_copy(ones, accn.at[idx_v], add=True)
            return 0

        lax.fori_loop(0, nfull, step, 0)
        if rem:
            (idx_r,) = rembufs
            base = base0 + nfull * C
            pltpu.sync_copy(idxb_ref.at[pl.ds(base, rem)], idx_r)
            pltpu.sync_copy(ones.at[pl.ds(0, rem)], accb.at[idx_r], add=True)
            pltpu.sync_copy(idxn_ref.at[pl.ds(base, rem)], idx_r)
            pltpu.sync_copy(ones.at[pl.ds(0, rem)], accn.at[idx_r], add=True)
        plsc.subcore_barrier()
        pltpu.sync_copy(accb.at[pl.ds(s * sob, sob)],
                        outb_ref.at[c, pl.ds(s * sob, sob)])
        pltpu.sync_copy(accn.at[pl.ds(s * son, son)],
                        outn_ref.at[c, pl.ds(s * son, son)])

    fn = pl.kernel(body,
                   out_type=[jax.ShapeDtypeStruct((NC, SB, H), jnp.float32),
                             jax.ShapeDtypeStruct((NC, SN, H), jnp.float32)],
                   mesh=_mesh(),
                   compiler_params=_SC_PARAMS,
                   scratch_types=scratch)
    return fn(idxb, idxn)


def _sc_seg_max(table, gidx, ids, S):
    N = gidx.shape[0]
    H = 16
    rows_pw = N // NW
    C = 128
    nfull, rem = divmod(rows_pw, C)
    scratch = [pltpu.VMEM((C,), jnp.int32),
               pltpu.VMEM((C,), jnp.int32),
               pltpu.VMEM((C, H), jnp.float32),
               pltpu.VMEM((S, H), jnp.float32),
               pltpu.SemaphoreType.DMA]
    if rem:
        scratch += [pltpu.VMEM((rem,), jnp.int32),
                    pltpu.VMEM((rem,), jnp.int32),
                    pltpu.VMEM((rem, H), jnp.float32)]

    def body(table_ref, gidx_ref, ids_ref, out_ref,
             gidx_v, ids_v, rows_v, tbl, sem, *rembufs):
        w = lax.axis_index("s") * NC + lax.axis_index("c")
        iota = lax.iota(jnp.int32, LANES)
        ninf = jnp.full((LANES,), -jnp.inf, jnp.float32)

        def init_row(r, _):
            plsc.store_scatter(tbl, [jnp.full((LANES,), r, jnp.int32), iota],
                               ninf)
            return 0

        lax.fori_loop(0, S, init_row, 0)

        def do_chunk(base, n, gv, iv, rv):
            pltpu.sync_copy(gidx_ref.at[pl.ds(base, n)], gv)
            pltpu.sync_copy(ids_ref.at[pl.ds(base, n)], iv)
            pltpu.async_copy(table_ref.at[gv], rv, sem).wait()

            def row(j, _):
                jfull = jnp.full((LANES,), j, jnp.int32)
                seg = plsc.load_gather(iv, [jfull])
                val = plsc.load_gather(rv, [jfull, iota])
                cur = plsc.load_gather(tbl, [seg, iota])
                plsc.store_scatter(tbl, [seg, iota], jnp.maximum(cur, val))
                return 0

            lax.fori_loop(0, n, row, 0)

        base0 = w * rows_pw

        def step(k, _):
            do_chunk(base0 + k * C, C, gidx_v, ids_v, rows_v)
            return 0

        lax.fori_loop(0, nfull, step, 0)
        if rem:
            gr, ir, rr = rembufs
            do_chunk(base0 + nfull * C, rem, gr, ir, rr)
        pltpu.sync_copy(tbl, out_ref.at[w])

    fn = pl.kernel(body,
                   out_type=jax.ShapeDtypeStruct((NW, S, H), jnp.float32),
                   mesh=_mesh(),
                   compiler_params=_SC_PARAMS,
                   scratch_types=scratch)
    return fn(table, gidx, ids)


def _full_spec(shape):
    nd = len(shape)
    return pl.BlockSpec(shape, lambda *_: (0,) * nd)


def _tc_linear_split(x, W, b, T):
    N, K = x.shape
    grid = (N // T,)

    def body(x_ref, w_ref, b_ref, ol, oh):
        r = jnp.dot(x_ref[...], w_ref[...],
                    preferred_element_type=jnp.float32) + b_ref[...]
        ol[...] = r[:, :HH]
        oh[...] = r[:, HH:]

    return pl.pallas_call(
        body, grid=grid,
        in_specs=[pl.BlockSpec((T, K), lambda i: (i, 0)),
                  _full_spec((K, NHID)), _full_spec((1, NHID))],
        out_specs=[pl.BlockSpec((T, HH), lambda i: (i, 0)),
                   pl.BlockSpec((T, HH), lambda i: (i, 0))],
        out_shape=[jax.ShapeDtypeStruct((N, HH), jnp.float32),
                   jax.ShapeDtypeStruct((N, HH), jnp.float32)])(x, W, b)


def _tc_gine_split(h_lo, h_hi, a_lo, a_hi, W1, b1, W2, b2):
    N = h_lo.shape[0]
    T = 632
    grid = (N // T,)

    def body(hl, hh_, al, ah, w1, b1_, w2, b2_, ol, oh):
        hv = jnp.concatenate([hl[...], hh_[...]], axis=1)
        u = hv + jnp.concatenate([al[...], ah[...]], axis=1)
        t = jnp.maximum(
            jnp.dot(u, w1[...], preferred_element_type=jnp.float32)
            + b1_[...], 0.0)
        v = jnp.dot(t, w2[...], preferred_element_type=jnp.float32) + b2_[...]
        r = jnp.maximum(v, 0.0) + hv
        ol[...] = r[:, :HH]
        oh[...] = r[:, HH:]

    rs = pl.BlockSpec((T, HH), lambda i: (i, 0))
    return pl.pallas_call(
        body, grid=grid,
        in_specs=[rs, rs, rs, rs,
                  _full_spec((NHID, NHID)), _full_spec((1, NHID)),
                  _full_spec((NHID, NHID)), _full_spec((1, NHID))],
        out_specs=[rs, rs],
        out_shape=[jax.ShapeDtypeStruct((N, HH), jnp.float32),
                   jax.ShapeDtypeStruct((N, HH), jnp.float32)])(
            h_lo, h_hi, a_lo, a_hi, W1, b1, W2, b2)


def _tc_mean_lin_relu_split(Ssum, cparts, W, b):
    S = Ssum.shape[0]

    def body(s_ref, c_ref, w_ref, b_ref, ol, oh):
        cp = c_ref[...]
        cnt = cp[0, :, :1] + cp[1, :, :1]
        m = s_ref[...] / jnp.maximum(cnt, 1.0)
        r = jnp.maximum(
            jnp.dot(m, w_ref[...], preferred_element_type=jnp.float32)
            + b_ref[...], 0.0)
        ol[...] = r[:, :HH]
        oh[...] = r[:, HH:]

    return pl.pallas_call(
        body,
        in_specs=[_full_spec((S, NHID)), _full_spec((NC, S, PRW)),
                  _full_spec((NHID, NHID)), _full_spec((1, NHID))],
        out_specs=[_full_spec((S, HH)), _full_spec((S, HH))],
        out_shape=[jax.ShapeDtypeStruct((S, HH), jnp.float32),
                   jax.ShapeDtypeStruct((S, HH), jnp.float32)])(
            Ssum, cparts, W, b)


def _tc_mean_split(Ssum, cparts):
    S = Ssum.shape[0]

    def body(s_ref, c_ref, ol, oh):
        cp = c_ref[...]
        cnt = cp[0, :, :1] + cp[1, :, :1]
        r = s_ref[...] / jnp.maximum(cnt, 1.0)
        ol[...] = r[:, :HH]
        oh[...] = r[:, HH:]

    return pl.pallas_call(
        body,
        in_specs=[_full_spec((S, NHID)), _full_spec((NC, S, PRW))],
        out_specs=[_full_spec((S, HH)), _full_spec((S, HH))],
        out_shape=[jax.ShapeDtypeStruct((S, HH), jnp.float32),
                   jax.ShapeDtypeStruct((S, HH), jnp.float32)])(Ssum, cparts)


def _ln(h, s, bb):
    m = h.mean(-1, keepdims=True)
    v = ((h - m) ** 2).mean(-1, keepdims=True)
    return (h - m) / jnp.sqrt(v + 1e-5) * s + bb


def _tc_tail(S3, cb2, pesp, ctx_ids, tgt_ids, flat):
    BT = B * NTGT

    def body(*refs):
        (s3, c2, pp, cid, tid,
         peW, peb,
         cWv, cbv, cWo, cbo, cf1, cb1, cf2, cb2_, cl1s, cl1b, cl2s, cl2b,
         tWq, tbq, tWk, tbk, tWv, tbv, tWo, tbo, tf1, tb1, tf2, tb2_,
         tl1s, tl1b, tl2s, tl2b,
         p1W, p1b, p2W, p2b, p3W, p3b,
         o1, o2) = refs

        cp = c2[...]
        cnt = cp[0, :, :1] + cp[1, :, :1]
        M3 = (s3[...] / jnp.maximum(cnt, 1.0))[:P_TOT]
        pmax = jnp.max(pp[...], axis=0)
        P = jnp.where(jnp.isfinite(pmax), pmax, 0.0)[:P_TOT]

        cidv = cid[...].reshape(B, 1)
        tidv = tid[...].reshape(BT, 1)
        oh_c = (lax.broadcasted_iota(jnp.int32, (B, P_TOT), 1)
                == cidv).astype(jnp.float32)
        oh_t = (lax.broadcasted_iota(jnp.int32, (BT, P_TOT), 1)
                == tidv).astype(jnp.float32)

        dotf = functools.partial(jnp.dot, preferred_element_type=jnp.float32)
        dotx = functools.partial(jnp.dot, preferred_element_type=jnp.float32,
                                 precision=lax.Precision.HIGHEST)
        cx = dotx(oh_c, M3)
        tx = dotx(oh_t, M3)
        cpe = dotx(oh_c, P)
        tpes = dotx(oh_t, P)

        cx = cx + dotf(cpe, peW[...]) + peb[...]
        enc_t = dotf(tpes, peW[...]) + peb[...]

        v = dotf(cx, cWv[...]) + cbv[...]
        cx = _ln(cx + dotf(v, cWo[...]) + cbo[...], cl1s[...], cl1b[...])
        f = dotf(jnp.maximum(dotf(cx, cf1[...]) + cb1[...], 0.0),
                 cf2[...]) + cb2_[...]
        cx = _ln(cx + f, cl2s[...], cl2b[...])

        q = dotf(tx, tWq[...]) + tbq[...]
        k = dotf(tx, tWk[...]) + tbk[...]
        vv = dotf(tx, tWv[...]) + tbv[...]
        ri = lax.broadcasted_iota(jnp.int32, (BT, BT), 0) // NTGT
        ci = lax.broadcasted_iota(jnp.int32, (BT, BT), 1) // NTGT
        blk = ri == ci
        dh = NHID // NHEADS
        outs = []
        for hd in range(NHEADS):
            sl = slice(hd * dh, (hd + 1) * dh)
            A = lax.dot_general(q[:, sl], k[:, sl],
                                (((1,), (1,)), ((), ())),
                                preferred_element_type=jnp.float32)
            A = A / jnp.sqrt(jnp.float32(dh))
            A = jnp.where(blk, A, -1e9)
            A = A - jnp.max(A, axis=-1, keepdims=True)
            E = jnp.exp(A)
            Pm = E / jnp.sum(E, axis=-1, keepdims=True)
            outs.append(dotf(Pm, vv[:, sl]))
        o = jnp.concatenate(outs, axis=1)
        tx = _ln(tx + dotf(o, tWo[...]) + tbo[...], tl1s[...], tl1b[...])
        f = dotf(jnp.maximum(dotf(tx, tf1[...]) + tb1[...], 0.0),
                 tf2[...]) + tb2_[...]
        tx = _ln(tx + f, tl2s[...], tl2b[...])

        m = jnp.mean(tx, axis=-1, keepdims=True)
        em, emi = jnp.exp(m), jnp.exp(-m)
        xo = jnp.concatenate([(em + emi) * 0.5, (em - emi) * 0.5], axis=1)
        o1[...] = jnp.concatenate(
            [xo, jnp.zeros((BT, NHID - 2), jnp.float32)], axis=1)

        rep = (lax.broadcasted_iota(jnp.int32, (BT, B), 0) // NTGT
               == lax.broadcasted_iota(jnp.int32, (BT, B), 1)
               ).astype(jnp.float32)
        tpe = dotx(rep, cx) + enc_t
        t1 = jnp.maximum(dotf(tpe, p1W[...]) + p1b[...], 0.0)
        t2 = jnp.maximum(dotf(t1, p2W[...]) + p2b[...], 0.0)
        o2[...] = dotf(t2, p3W[...]) + p3b[...]

    specs = [_full_spec(a.shape) for a in flat]
    return pl.pallas_call(
        body,
        in_specs=[_full_spec(S3.shape), _full_spec(cb2.shape),
                  _full_spec(pesp.shape), _full_spec(ctx_ids.shape),
                  _full_spec(tgt_ids.shape)] + specs,
        out_specs=[_full_spec((BT, NHID)), _full_spec((BT, NHID))],
        out_shape=[jax.ShapeDtypeStruct((BT, NHID), jnp.float32),
                   jax.ShapeDtypeStruct((BT, NHID), jnp.float32)])(
            S3, cb2, pesp, ctx_ids, tgt_ids, *flat)


def _rb(b):
    return b.reshape(1, -1)


def kernel(x, edge_attr, rw_pos_enc, combined_subgraphs,
           subgraphs_nodes_mapper, subgraphs_edges_mapper, subgraphs_batch,
           context_subgraph_idx, target_subgraph_idxs, mask, params):
    p = params
    i32 = jnp.int32
    nm = subgraphs_nodes_mapper.astype(i32)
    em = subgraphs_edges_mapper.astype(i32)
    bx = subgraphs_batch.astype(i32)
    src = combined_subgraphs[0].astype(i32)
    dst = combined_subgraphs[1].astype(i32)

    padn = NP_SUB - N_SUB
    nm_g = jnp.concatenate([nm, jnp.zeros((padn,), i32)])
    nm_s = jnp.concatenate([nm, jnp.full((padn,), N_NODES, i32)])
    bx_s = jnp.concatenate([bx, jnp.full((padn,), P_TOT, i32)])
    h0_lo, h0_hi = _tc_linear_split(x, p["inp"]["W"], _rb(p["inp"]["b"]),
                                    2000)
    ea16 = _sc_gather(edge_attr, em, PRW)
    e_lo, e_hi = _tc_linear_split(ea16, p["edge"]["W"], _rb(p["edge"]["b"]),
                                  512)
    h_lo, h_hi = _sc_gather_half(h0_lo, h0_hi, nm_g)
    pesp = _sc_seg_max(rw_pos_enc, nm_g, bx_s, SB)
    cb2, cn2 = _sc_counts(bx_s, nm_s)

    a_lo, a_hi = _sc_gine_msg(e_lo, e_hi, h_lo, h_hi, src, dst)
    g = p["gnn"][0]
    h_lo, h_hi = _tc_gine_split(h_lo, h_hi, a_lo, a_hi,
                                g["l1"]["W"], _rb(g["l1"]["b"]),
                                g["l2"]["W"], _rb(g["l2"]["b"]))

    S1 = _sc_scatter_add_split(h_lo, h_hi, bx_s, SB)
    R_lo, R_hi = _tc_mean_lin_relu_split(S1, cb2, p["U"]["W"],
                                         _rb(p["U"]["b"]))
    S2 = _sc_mix_scatter(R_lo, R_hi, bx_s, h_lo, h_hi, nm_s, SN)
    M2_lo, M2_hi = _tc_mean_split(S2, cn2)
    h_lo, h_hi = _sc_gather_half(M2_lo, M2_hi, nm_g)

    a_lo, a_hi = _sc_gine_msg(e_lo, e_hi, h_lo, h_hi, src, dst)
    g = p["gnn"][1]
    h_lo, h_hi = _tc_gine_split(h_lo, h_hi, a_lo, a_hi,
                                g["l1"]["W"], _rb(g["l1"]["b"]),
                                g["l2"]["W"], _rb(g["l2"]["b"]))

    S3 = _sc_scatter_add_split(h_lo, h_hi, bx_s, SB)

    batch_indexer = jnp.arange(B, dtype=i32) * NPATCH
    ctx_ids = (context_subgraph_idx.astype(i32) + batch_indexer).reshape(1, B)
    tgt_ids = (target_subgraph_idxs.astype(i32)
               + batch_indexer[:, None]).reshape(1, B * NTGT)

    c = p["ctx"][0]
    t = p["tgt"][0]
    p1, p2, p3 = p["pred"]
    p3W = jnp.zeros((NHID, NHID), jnp.float32).at[:, :2].set(p3["W"])
    p3b = jnp.zeros((1, NHID), jnp.float32).at[:, :2].set(_rb(p3["b"]))
    flat = [
        p["pe"]["W"], _rb(p["pe"]["b"]),
        c["Wv"]["W"], _rb(c["Wv"]["b"]), c["Wo"]["W"], _rb(c["Wo"]["b"]),
        c["ff1"]["W"], _rb(c["ff1"]["b"]), c["ff2"]["W"], _rb(c["ff2"]["b"]),
        _rb(c["ln1"]["s"]), _rb(c["ln1"]["b"]),
        _rb(c["ln2"]["s"]), _rb(c["ln2"]["b"]),
        t["Wq"]["W"], _rb(t["Wq"]["b"]), t["Wk"]["W"], _rb(t["Wk"]["b"]),
        t["Wv"]["W"], _rb(t["Wv"]["b"]), t["Wo"]["W"], _rb(t["Wo"]["b"]),
        t["ff1"]["W"], _rb(t["ff1"]["b"]), t["ff2"]["W"], _rb(t["ff2"]["b"]),
        _rb(t["ln1"]["s"]), _rb(t["ln1"]["b"]),
        _rb(t["ln2"]["s"]), _rb(t["ln2"]["b"]),
        p1["W"], _rb(p1["b"]), p2["W"], _rb(p2["b"]), p3W, p3b,
    ]
    o1, o2 = _tc_tail(S3, cb2, pesp, ctx_ids, tgt_ids, flat)

    target_x_out = o1[:, :2].reshape(B, NTGT, 2)
    target_y = o2[:, :2].reshape(B, NTGT, 2)
    return (target_x_out, target_y)
